# Initial kernel scaffold; baseline (speedup 1.0000x reference)
import jax
import jax.numpy as jnp
from jax.experimental import pallas as pl


def kernel(x, edge_index, pre_W, pre_b, post_W, post_b, lin_W, lin_b):
    raise NotImplementedError("write your pallas kernel here")


# speedup vs baseline: 1.2241x; 1.2241x over previous
import functools

import jax
import jax.numpy as jnp
import numpy as np
from jax.experimental import pallas as pl

_N = 10000
_E = 320000
_F = 128
_LAYERS = 3
_ADL = 0.0

_BN = 1000


def _pre_body(h_ref, wd_ref, ws_ref, pb_ref, a_ref, b_ref):
    h = h_ref[...]
    a_ref[...] = jnp.dot(h, wd_ref[...], preferred_element_type=jnp.float32) + pb_ref[...]
    b_ref[...] = jnp.dot(h, ws_ref[...], preferred_element_type=jnp.float32)


def _pre_stage(h, pW, pb):
    wd, ws = pW[:_F], pW[_F:]
    grid = (_N // _BN,)
    a, b = pl.pallas_call(
        _pre_body,
        grid=grid,
        in_specs=[
            pl.BlockSpec((_BN, _F), lambda i: (i, 0)),
            pl.BlockSpec((_F, _F), lambda i: (0, 0)),
            pl.BlockSpec((_F, _F), lambda i: (0, 0)),
            pl.BlockSpec((1, _F), lambda i: (0, 0)),
        ],
        out_specs=[
            pl.BlockSpec((_BN, _F), lambda i: (i, 0)),
            pl.BlockSpec((_BN, _F), lambda i: (i, 0)),
        ],
        out_shape=[
            jax.ShapeDtypeStruct((_N, _F), jnp.float32),
            jax.ShapeDtypeStruct((_N, _F), jnp.float32),
        ],
    )(h, wd, ws, pb[None, :])
    return a, b


def _post_body(h_ref, a_ref, s1_ref, s2_ref, mx_ref, mn_ref, cnt_ref,
               pow_ref, pob_ref, lw_ref, lb_ref, out_ref):
    h = h_ref[...]
    a = a_ref[...]
    s1 = s1_ref[...]
    s2 = s2_ref[...]
    cnt = cnt_ref[...]
    cc = jnp.maximum(cnt, 1.0)
    mean = (cnt * a + s1) / cc
    mean2 = (cnt * a * a + 2.0 * a * s1 + s2) / cc
    var = mean2 - mean * mean
    std = jnp.sqrt(jnp.maximum(var, 1e-5))
    std = jnp.where(std <= np.sqrt(1e-5), 0.0, std)
    has = cnt > 0
    mx = jnp.where(has, a + mx_ref[...], 0.0)
    mn = jnp.where(has, a + mn_ref[...], 0.0)
    agg = jnp.concatenate([mean, mx, mn, std], axis=-1)
    log_deg = jnp.log(jnp.maximum(cnt, 1.0) + 1.0)
    ld4 = jnp.concatenate([log_deg] * 4, axis=-1)
    amp = agg * (ld4 / _ADL)
    att = agg * (_ADL / ld4)
    cat = jnp.concatenate([h, agg, amp, att], axis=-1)
    o = jnp.dot(cat, pow_ref[...], preferred_element_type=jnp.float32) + pob_ref[...]
    o = jnp.dot(o, lw_ref[...], preferred_element_type=jnp.float32) + lb_ref[...]
    out_ref[...] = o + h


def _post_stage(h, a, s1, s2, mx, mn, cnt, poW, pob, lW, lb):
    grid = (_N // _BN,)
    nf = pl.BlockSpec((_BN, _F), lambda i: (i, 0))
    full = lambda r, c: pl.BlockSpec((r, c), lambda i: (0, 0))
    return pl.pallas_call(
        _post_body,
        grid=grid,
        in_specs=[nf, nf, nf, nf, nf, nf, nf,
                  full(13 * _F, _F), full(1, _F), full(_F, _F), full(1, _F)],
        out_specs=nf,
        out_shape=jax.ShapeDtypeStruct((_N, _F), jnp.float32),
    )(h, a, s1, s2, mx, mn, cnt, poW, pob[None, :], lW, lb[None, :])


def kernel(x, edge_index, pre_W, pre_b, post_W, post_b, lin_W, lin_b):
    src, dst = edge_index[0], edge_index[1]
    h = x
    for i in range(_LAYERS):
        a, b = _pre_stage(h, pre_W[i], pre_b[i])
        bs = b[src]
        s1 = jax.ops.segment_sum(bs, dst, num_segments=_N)
        s2 = jax.ops.segment_sum(bs * bs, dst, num_segments=_N)
        gmx = jax.ops.segment_max(bs, dst, num_segments=_N)
        gmn = jax.ops.segment_min(bs, dst, num_segments=_N)
        cnt = jax.ops.segment_sum(jnp.ones((_E,), jnp.float32), dst, num_segments=_N)
        cnt128 = jnp.broadcast_to(cnt[:, None], (_N, _F))
        h = _post_stage(h, a, s1, s2, gmx, gmn, cnt128, post_W[i], post_b[i],
                        lin_W[i], lin_b[i])
    return h


# speedup vs baseline: 1.4664x; 1.1980x over previous
import functools

import jax
import jax.numpy as jnp
import numpy as np
from jax import lax
from jax.experimental import pallas as pl
from jax.experimental.pallas import tpu as pltpu, tpu_sc as plsc

_N = 10000
_NP = 10240
_E = 320000
_F = 128
_LAYERS = 3
_ADL = 0.0

_BN = 1024
_CH = 2000
_CPS = 4
_NSUB = 32

_sc_mesh = plsc.VectorSubcoreMesh(core_axis_name="c", subcore_axis_name="s")


def _pre_body(ht_ref, wd_ref, ws_ref, pb_ref, at_ref, bt_ref):
    ht = ht_ref[...]
    dn = (((0,), (0,)), ((), ()))
    at_ref[...] = (lax.dot_general(wd_ref[...], ht, dn,
                                   preferred_element_type=jnp.float32)
                   + pb_ref[...])
    bt_ref[...] = lax.dot_general(ws_ref[...], ht, dn,
                                  preferred_element_type=jnp.float32)


def _pre_stage(ht, pW, pb):
    wd, ws = pW[:_F], pW[_F:]
    fn = pl.BlockSpec((_F, _BN), lambda i: (0, i))
    ff = pl.BlockSpec((_F, _F), lambda i: (0, 0))
    return pl.pallas_call(
        _pre_body,
        grid=(_NP // _BN,),
        in_specs=[fn, ff, ff, pl.BlockSpec((_F, 1), lambda i: (0, 0))],
        out_specs=[fn, fn],
        out_shape=[jax.ShapeDtypeStruct((_F, _NP), jnp.float32),
                   jax.ShapeDtypeStruct((_F, _NP), jnp.float32)],
    )(ht, wd, ws, pb[:, None])


@functools.partial(
    pl.kernel, mesh=_sc_mesh,
    compiler_params=pltpu.CompilerParams(needs_layout_passes=False),
    out_type=[jax.ShapeDtypeStruct((_F * _NP,), jnp.float32),
              jax.ShapeDtypeStruct((_F * _NP,), jnp.float32),
              jax.ShapeDtypeStruct((_F * _NP,), jnp.float32),
              jax.ShapeDtypeStruct((_F * _NP,), jnp.float32),
              jax.ShapeDtypeStruct((_NSUB * _NP,), jnp.float32)],
    scratch_types=[pltpu.VMEM((_CPS * _NP,), jnp.float32),
                   pltpu.VMEM((_CPS * _NP,), jnp.float32),
                   pltpu.VMEM((_CPS * _NP,), jnp.float32),
                   pltpu.VMEM((_CH,), jnp.int32),
                   pltpu.VMEM((_CH,), jnp.int32)])
def _sc_edge(bt, srcl, dstl, s1, s2, smx, smn, cntp,
             tbl, acc0, acc1, esrc, edst):
    cid = lax.axis_index("c")
    sid = lax.axis_index("s")
    wid = sid * 2 + cid
    col0 = wid * _CPS

    pltpu.sync_copy(bt.at[pl.ds(col0 * _NP, _CPS * _NP)], tbl)

    zeros = jnp.zeros((16,), jnp.float32)

    def _init(val0, val1):
        def body(j, _):
            acc0[pl.ds(j * 16, 16)] = val0
            acc1[pl.ds(j * 16, 16)] = val1
            return 0
        lax.fori_loop(0, (_CPS * _NP) // 16, body, 0)

    _init(zeros, zeros)

    def chunk_a(k, _):
        c0 = k * _CH
        pltpu.sync_copy(srcl.at[pl.ds(c0, _CH)], esrc)
        pltpu.sync_copy(dstl.at[pl.ds(c0, _CH)], edst)

        def group(g, _):
            s16 = esrc[pl.ds(g * 16, 16)]
            d16 = edst[pl.ds(g * 16, 16)]
            for c in range(_CPS):
                b = plsc.load_gather(tbl, [s16 + (c * _NP)])
                offd = d16 + (c * _NP)
                plsc.addupdate_scatter(acc0, [offd], b)
                plsc.addupdate_scatter(acc1, [offd], b * b)
            return 0
        lax.fori_loop(0, _CH // 16, group, 0)
        return 0
    lax.fori_loop(0, _E // _CH, chunk_a, 0)
    pltpu.sync_copy(acc0, s1.at[pl.ds(col0 * _NP, _CPS * _NP)])
    pltpu.sync_copy(acc1, s2.at[pl.ds(col0 * _NP, _CPS * _NP)])

    _init(jnp.full((16,), -jnp.inf, jnp.float32),
          jnp.full((16,), jnp.inf, jnp.float32))

    def chunk_b(k, _):
        c0 = k * _CH
        pltpu.sync_copy(srcl.at[pl.ds(c0, _CH)], esrc)
        pltpu.sync_copy(dstl.at[pl.ds(c0, _CH)], edst)

        def group(g, _):
            s16 = esrc[pl.ds(g * 16, 16)]
            d16 = edst[pl.ds(g * 16, 16)]
            bs = [plsc.load_gather(tbl, [s16 + (c * _NP)]) for c in range(_CPS)]

            def rmw(_):
                pend = jnp.zeros((16,), jnp.bool_)
                for c in range(_CPS):
                    offd = d16 + (c * _NP)
                    b = bs[c]
                    cur = plsc.load_gather(acc0, [offd])
                    plsc.store_scatter(acc0, [offd], jnp.maximum(cur, b),
                                       mask=cur < b)
                    pend = pend | (plsc.load_gather(acc0, [offd]) < b)
                    curn = plsc.load_gather(acc1, [offd])
                    plsc.store_scatter(acc1, [offd], jnp.minimum(curn, b),
                                       mask=curn > b)
                    pend = pend | (plsc.load_gather(acc1, [offd]) > b)
                return jnp.any(pend)
            lax.while_loop(lambda p: p, rmw, jnp.bool_(True))
            return 0
        lax.fori_loop(0, _CH // 16, group, 0)
        return 0
    lax.fori_loop(0, _E // _CH, chunk_b, 0)
    pltpu.sync_copy(acc0, smx.at[pl.ds(col0 * _NP, _CPS * _NP)])
    pltpu.sync_copy(acc1, smn.at[pl.ds(col0 * _NP, _CPS * _NP)])

    def zcnt(j, _):
        acc0[pl.ds(j * 16, 16)] = zeros
        return 0
    lax.fori_loop(0, _NP // 16, zcnt, 0)
    ones = jnp.ones((16,), jnp.float32)
    eper = _E // _NSUB

    def chunk_c(k, _):
        c0 = wid * eper + k * _CH
        pltpu.sync_copy(dstl.at[pl.ds(c0, _CH)], edst)

        def group(g, _):
            plsc.addupdate_scatter(acc0, [edst[pl.ds(g * 16, 16)]], ones)
            return 0
        lax.fori_loop(0, _CH // 16, group, 0)
        return 0
    lax.fori_loop(0, eper // _CH, chunk_c, 0)
    pltpu.sync_copy(acc0.at[pl.ds(0, _NP)], cntp.at[pl.ds(wid * _NP, _NP)])


def _post_body(ht_ref, at_ref, s1_ref, s2_ref, mx_ref, mn_ref, cntp_ref,
               pow_ref, pob_ref, lw_ref, lb_ref, out_ref):
    ht = ht_ref[...]
    a = at_ref[...]
    s1 = s1_ref[...]
    s2 = s2_ref[...]
    cnt = jnp.sum(cntp_ref[...], axis=0, keepdims=True)
    cc = jnp.maximum(cnt, 1.0)
    mean = (cnt * a + s1) / cc
    mean2 = (cnt * a * a + 2.0 * a * s1 + s2) / cc
    var = mean2 - mean * mean
    std = jnp.sqrt(jnp.maximum(var, 1e-5))
    std = jnp.where(std <= np.sqrt(1e-5), 0.0, std)
    has = cnt > 0
    mxv = jnp.where(has, a + mx_ref[...], 0.0)
    mnv = jnp.where(has, a + mn_ref[...], 0.0)
    agg = jnp.concatenate([mean, mxv, mnv, std], axis=0)
    log_deg = jnp.log(jnp.maximum(cnt, 1.0) + 1.0)
    amp = agg * (log_deg / _ADL)
    att = agg * (_ADL / log_deg)
    cat = jnp.concatenate([ht, agg, amp, att], axis=0)
    dn = (((0,), (0,)), ((), ()))
    o = (lax.dot_general(pow_ref[...], cat, dn,
                         preferred_element_type=jnp.float32) + pob_ref[...])
    o = (lax.dot_general(lw_ref[...], o, dn,
                         preferred_element_type=jnp.float32) + lb_ref[...])
    out_ref[...] = o + ht


def _post_stage(ht, at, s1, s2, mx, mn, cntp, poW, pob, lW, lb):
    fn = pl.BlockSpec((_F, _BN), lambda i: (0, i))
    return pl.pallas_call(
        _post_body,
        grid=(_NP // _BN,),
        in_specs=[fn, fn, fn, fn, fn, fn,
                  pl.BlockSpec((_NSUB, _BN), lambda i: (0, i)),
                  pl.BlockSpec((13 * _F, _F), lambda i: (0, 0)),
                  pl.BlockSpec((_F, 1), lambda i: (0, 0)),
                  pl.BlockSpec((_F, _F), lambda i: (0, 0)),
                  pl.BlockSpec((_F, 1), lambda i: (0, 0))],
        out_specs=fn,
        out_shape=jax.ShapeDtypeStruct((_F, _NP), jnp.float32),
    )(ht, at, s1, s2, mx, mn, cntp, poW, pob[:, None], lW, lb[:, None])


def kernel(x, edge_index, pre_W, pre_b, post_W, post_b, lin_W, lin_b):
    src, dst = edge_index[0], edge_index[1]
    ht = jnp.pad(x.T, ((0, 0), (0, _NP - _N)))
    for i in range(_LAYERS):
        at, bt = _pre_stage(ht, pre_W[i], pre_b[i])
        s1, s2, smx, smn, cntp = _sc_edge(bt.reshape(_F * _NP), src, dst)
        ht = _post_stage(ht, at,
                         s1.reshape(_F, _NP), s2.reshape(_F, _NP),
                         smx.reshape(_F, _NP), smn.reshape(_F, _NP),
                         cntp.reshape(_NSUB, _NP),
                         post_W[i], post_b[i], lin_W[i], lin_b[i])
    return ht[:, :_N].T


# speedup vs baseline: 1.6293x; 1.1111x over previous
import functools

import jax
import jax.numpy as jnp
import numpy as np
from jax import lax
from jax.experimental import pallas as pl
from jax.experimental.pallas import tpu as pltpu, tpu_sc as plsc

_N = 10000
_NP = 10240
_E = 320000
_F = 128
_LAYERS = 3
_ADL = 0.0

_BN = 1024
_CH = 2000
_CPS = 4
_NSUB = 32

_sc_mesh = plsc.VectorSubcoreMesh(core_axis_name="c", subcore_axis_name="s")


def _pre_body(ht_ref, wd_ref, ws_ref, pb_ref, at_ref, bt_ref):
    ht = ht_ref[...]
    dn = (((0,), (0,)), ((), ()))
    at_ref[...] = (lax.dot_general(wd_ref[...], ht, dn,
                                   preferred_element_type=jnp.float32)
                   + pb_ref[...])
    bt_ref[...] = lax.dot_general(ws_ref[...], ht, dn,
                                  preferred_element_type=jnp.float32)


def _pre_stage(ht, pW, pb):
    wd, ws = pW[:_F], pW[_F:]
    fn = pl.BlockSpec((_F, _BN), lambda i: (0, i))
    ff = pl.BlockSpec((_F, _F), lambda i: (0, 0))
    return pl.pallas_call(
        _pre_body,
        grid=(_NP // _BN,),
        in_specs=[fn, ff, ff, pl.BlockSpec((_F, 1), lambda i: (0, 0))],
        out_specs=[fn, fn],
        out_shape=[jax.ShapeDtypeStruct((_F, _NP), jnp.float32),
                   jax.ShapeDtypeStruct((_F, _NP), jnp.float32)],
    )(ht, wd, ws, pb[:, None])


@functools.partial(
    pl.kernel, mesh=_sc_mesh,
    compiler_params=pltpu.CompilerParams(needs_layout_passes=False),
    out_type=[jax.ShapeDtypeStruct((_F * _NP,), jnp.float32),
              jax.ShapeDtypeStruct((_F * _NP,), jnp.float32),
              jax.ShapeDtypeStruct((_F * _NP,), jnp.float32),
              jax.ShapeDtypeStruct((_F * _NP,), jnp.float32),
              jax.ShapeDtypeStruct((_NSUB * _NP,), jnp.float32)],
    scratch_types=[pltpu.VMEM((_CPS * _NP,), jnp.float32),
                   pltpu.VMEM((_CPS * _NP,), jnp.float32),
                   pltpu.VMEM((_CPS * _NP,), jnp.float32),
                   pltpu.VMEM((_CH,), jnp.int32),
                   pltpu.VMEM((_CH,), jnp.int32)])
def _sc_edge(bt, srcl, dstl, s1, s2, smx, smn, cntp,
             tbl, acc0, acc1, esrc, edst):
    cid = lax.axis_index("c")
    sid = lax.axis_index("s")
    wid = sid * 2 + cid
    col0 = wid * _CPS

    pltpu.sync_copy(bt.at[pl.ds(col0 * _NP, _CPS * _NP)], tbl)

    zeros = jnp.zeros((16,), jnp.float32)

    def _init(val0, val1):
        def body(j, _):
            acc0[pl.ds(j * 16, 16)] = val0
            acc1[pl.ds(j * 16, 16)] = val1
            return 0
        lax.fori_loop(0, (_CPS * _NP) // 16, body, 0)

    _init(zeros, zeros)

    def chunk_a(k, _):
        c0 = k * _CH
        pltpu.sync_copy(srcl.at[pl.ds(c0, _CH)], esrc)
        pltpu.sync_copy(dstl.at[pl.ds(c0, _CH)], edst)

        def group(g):
            s16 = esrc[pl.ds(g * 16, 16)]
            d16 = edst[pl.ds(g * 16, 16)]
            for c in range(_CPS):
                b = plsc.load_gather(tbl, [s16 + (c * _NP)])
                offd = d16 + (c * _NP)
                plsc.addupdate_scatter(acc0, [offd], b)
                plsc.addupdate_scatter(acc1, [offd], b * b)
        plsc.parallel_loop(0, _CH // 16, unroll=8)(group)
        return 0
    lax.fori_loop(0, _E // _CH, chunk_a, 0)
    pltpu.sync_copy(acc0, s1.at[pl.ds(col0 * _NP, _CPS * _NP)])
    pltpu.sync_copy(acc1, s2.at[pl.ds(col0 * _NP, _CPS * _NP)])

    _init(jnp.full((16,), -jnp.inf, jnp.float32),
          jnp.full((16,), jnp.inf, jnp.float32))

    def chunk_b(k, _):
        c0 = k * _CH
        pltpu.sync_copy(srcl.at[pl.ds(c0, _CH)], esrc)
        pltpu.sync_copy(dstl.at[pl.ds(c0, _CH)], edst)

        def group(g, _):
            s16 = esrc[pl.ds(g * 16, 16)]
            d16 = edst[pl.ds(g * 16, 16)]
            offds = [d16 + (c * _NP) for c in range(_CPS)]
            bs = [plsc.load_gather(tbl, [s16 + (c * _NP)]) for c in range(_CPS)]
            for c in range(_CPS):
                cur = plsc.load_gather(acc0, [offds[c]])
                plsc.store_scatter(acc0, [offds[c]], jnp.maximum(cur, bs[c]),
                                   mask=cur < bs[c])
                curn = plsc.load_gather(acc1, [offds[c]])
                plsc.store_scatter(acc1, [offds[c]], jnp.minimum(curn, bs[c]),
                                   mask=curn > bs[c])
            pend = jnp.zeros((16,), jnp.bool_)
            for c in range(_CPS):
                pend = pend | (plsc.load_gather(acc0, [offds[c]]) < bs[c])
                pend = pend | (plsc.load_gather(acc1, [offds[c]]) > bs[c])

            @pl.when(jnp.any(pend))
            def _retry():
                def rmw(_):
                    p2 = jnp.zeros((16,), jnp.bool_)
                    for c in range(_CPS):
                        b = bs[c]
                        cur = plsc.load_gather(acc0, [offds[c]])
                        plsc.store_scatter(acc0, [offds[c]],
                                           jnp.maximum(cur, b), mask=cur < b)
                        p2 = p2 | (plsc.load_gather(acc0, [offds[c]]) < b)
                        curn = plsc.load_gather(acc1, [offds[c]])
                        plsc.store_scatter(acc1, [offds[c]],
                                           jnp.minimum(curn, b), mask=curn > b)
                        p2 = p2 | (plsc.load_gather(acc1, [offds[c]]) > b)
                    return jnp.any(p2)
                lax.while_loop(lambda p: p, rmw, jnp.bool_(True))
            return 0
        lax.fori_loop(0, _CH // 16, group, 0)
        return 0
    lax.fori_loop(0, _E // _CH, chunk_b, 0)
    pltpu.sync_copy(acc0, smx.at[pl.ds(col0 * _NP, _CPS * _NP)])
    pltpu.sync_copy(acc1, smn.at[pl.ds(col0 * _NP, _CPS * _NP)])

    def zcnt(j, _):
        acc0[pl.ds(j * 16, 16)] = zeros
        return 0
    lax.fori_loop(0, _NP // 16, zcnt, 0)
    ones = jnp.ones((16,), jnp.float32)
    eper = _E // _NSUB

    def chunk_c(k, _):
        c0 = wid * eper + k * _CH
        pltpu.sync_copy(dstl.at[pl.ds(c0, _CH)], edst)

        def group(g):
            plsc.addupdate_scatter(acc0, [edst[pl.ds(g * 16, 16)]], ones)
        plsc.parallel_loop(0, _CH // 16, unroll=8)(group)
        return 0
    lax.fori_loop(0, eper // _CH, chunk_c, 0)
    pltpu.sync_copy(acc0.at[pl.ds(0, _NP)], cntp.at[pl.ds(wid * _NP, _NP)])


def _post_body(ht_ref, at_ref, s1_ref, s2_ref, mx_ref, mn_ref, cntp_ref,
               pow_ref, pob_ref, lw_ref, lb_ref, out_ref):
    ht = ht_ref[...]
    a = at_ref[...]
    s1 = s1_ref[...]
    s2 = s2_ref[...]
    cnt = jnp.sum(cntp_ref[...], axis=0, keepdims=True)
    cc = jnp.maximum(cnt, 1.0)
    mean = (cnt * a + s1) / cc
    mean2 = (cnt * a * a + 2.0 * a * s1 + s2) / cc
    var = mean2 - mean * mean
    std = jnp.sqrt(jnp.maximum(var, 1e-5))
    std = jnp.where(std <= np.sqrt(1e-5), 0.0, std)
    has = cnt > 0
    mxv = jnp.where(has, a + mx_ref[...], 0.0)
    mnv = jnp.where(has, a + mn_ref[...], 0.0)
    agg = jnp.concatenate([mean, mxv, mnv, std], axis=0)
    log_deg = jnp.log(jnp.maximum(cnt, 1.0) + 1.0)
    amp = agg * (log_deg / _ADL)
    att = agg * (_ADL / log_deg)
    cat = jnp.concatenate([ht, agg, amp, att], axis=0)
    dn = (((0,), (0,)), ((), ()))
    o = (lax.dot_general(pow_ref[...], cat, dn,
                         preferred_element_type=jnp.float32) + pob_ref[...])
    o = (lax.dot_general(lw_ref[...], o, dn,
                         preferred_element_type=jnp.float32) + lb_ref[...])
    out_ref[...] = o + ht


def _post_stage(ht, at, s1, s2, mx, mn, cntp, poW, pob, lW, lb):
    fn = pl.BlockSpec((_F, _BN), lambda i: (0, i))
    return pl.pallas_call(
        _post_body,
        grid=(_NP // _BN,),
        in_specs=[fn, fn, fn, fn, fn, fn,
                  pl.BlockSpec((_NSUB, _BN), lambda i: (0, i)),
                  pl.BlockSpec((13 * _F, _F), lambda i: (0, 0)),
                  pl.BlockSpec((_F, 1), lambda i: (0, 0)),
                  pl.BlockSpec((_F, _F), lambda i: (0, 0)),
                  pl.BlockSpec((_F, 1), lambda i: (0, 0))],
        out_specs=fn,
        out_shape=jax.ShapeDtypeStruct((_F, _NP), jnp.float32),
    )(ht, at, s1, s2, mx, mn, cntp, poW, pob[:, None], lW, lb[:, None])


def kernel(x, edge_index, pre_W, pre_b, post_W, post_b, lin_W, lin_b):
    src, dst = edge_index[0], edge_index[1]
    ht = jnp.pad(x.T, ((0, 0), (0, _NP - _N)))
    for i in range(_LAYERS):
        at, bt = _pre_stage(ht, pre_W[i], pre_b[i])
        s1, s2, smx, smn, cntp = _sc_edge(bt.reshape(_F * _NP), src, dst)
        ht = _post_stage(ht, at,
                         s1.reshape(_F, _NP), s2.reshape(_F, _NP),
                         smx.reshape(_F, _NP), smn.reshape(_F, _NP),
                         cntp.reshape(_NSUB, _NP),
                         post_W[i], post_b[i], lin_W[i], lin_b[i])
    return ht[:, :_N].T


# speedup vs baseline: 2.5064x; 1.5383x over previous
import functools

import jax
import jax.numpy as jnp
import numpy as np
from jax import lax
from jax.experimental import pallas as pl
from jax.experimental.pallas import tpu as pltpu, tpu_sc as plsc

_N = 10000
_NP = 10240
_E = 320000
_F = 128
_LAYERS = 3
_ADL = 0.0

_BN = 1024
_CH = 2000
_CPS = 4
_NSUB = 32

_sc_mesh = plsc.VectorSubcoreMesh(core_axis_name="c", subcore_axis_name="s")


def _pre_body(ht_ref, wd_ref, ws_ref, pb_ref, at_ref, bt_ref):
    ht = ht_ref[...]
    dn = (((0,), (0,)), ((), ()))
    at_ref[...] = (lax.dot_general(wd_ref[...], ht, dn,
                                   preferred_element_type=jnp.float32)
                   + pb_ref[...])
    bt_ref[...] = lax.dot_general(ws_ref[...], ht, dn,
                                  preferred_element_type=jnp.float32)


def _pre_stage(ht, pW, pb):
    wd, ws = pW[:_F], pW[_F:]
    fn = pl.BlockSpec((_F, _BN), lambda i: (0, i))
    ff = pl.BlockSpec((_F, _F), lambda i: (0, 0))
    return pl.pallas_call(
        _pre_body,
        grid=(_NP // _BN,),
        in_specs=[fn, ff, ff, pl.BlockSpec((_F, 1), lambda i: (0, 0))],
        out_specs=[fn, fn],
        out_shape=[jax.ShapeDtypeStruct((_F, _NP), jnp.float32),
                   jax.ShapeDtypeStruct((_F, _NP), jnp.float32)],
    )(ht, wd, ws, pb[:, None])


@functools.partial(
    pl.kernel, mesh=_sc_mesh,
    compiler_params=pltpu.CompilerParams(needs_layout_passes=False),
    out_type=[jax.ShapeDtypeStruct((_F * _NP,), jnp.float32),
              jax.ShapeDtypeStruct((_F * _NP,), jnp.float32),
              jax.ShapeDtypeStruct((_F * _NP,), jnp.float32),
              jax.ShapeDtypeStruct((_F * _NP,), jnp.float32),
              jax.ShapeDtypeStruct((_NSUB * _NP,), jnp.float32)],
    scratch_types=[pltpu.VMEM((_CPS * _NP,), jnp.float32),
                   pltpu.VMEM((_CPS * _NP,), jnp.float32),
                   pltpu.VMEM((_CPS * _NP,), jnp.float32),
                   pltpu.VMEM((_CH,), jnp.int32),
                   pltpu.VMEM((_CH,), jnp.int32),
                   pltpu.VMEM((_CH,), jnp.int32)])
def _sc_edge(bt, srcl, dstl, s1, s2, smx, smn, cntp,
             tbl, acc0, acc1, esrc, edst, flags):
    cid = lax.axis_index("c")
    sid = lax.axis_index("s")
    wid = sid * 2 + cid
    col0 = wid * _CPS

    pltpu.sync_copy(bt.at[pl.ds(col0 * _NP, _CPS * _NP)], tbl)

    zeros = jnp.zeros((16,), jnp.float32)

    def _init(val0, val1):
        def body(j, _):
            acc0[pl.ds(j * 16, 16)] = val0
            acc1[pl.ds(j * 16, 16)] = val1
            return 0
        lax.fori_loop(0, (_CPS * _NP) // 16, body, 0)

    _init(zeros, zeros)

    def chunk_a(k, _):
        c0 = k * _CH
        pltpu.sync_copy(srcl.at[pl.ds(c0, _CH)], esrc)
        pltpu.sync_copy(dstl.at[pl.ds(c0, _CH)], edst)

        def group(g):
            s16 = esrc[pl.ds(g * 16, 16)]
            d16 = edst[pl.ds(g * 16, 16)]
            for c in range(_CPS):
                b = plsc.load_gather(tbl, [s16 + (c * _NP)])
                offd = d16 + (c * _NP)
                plsc.addupdate_scatter(acc0, [offd], b)
                plsc.addupdate_scatter(acc1, [offd], b * b)
        plsc.parallel_loop(0, _CH // 16, unroll=8)(group)
        return 0
    lax.fori_loop(0, _E // _CH, chunk_a, 0)
    pltpu.sync_copy(acc0, s1.at[pl.ds(col0 * _NP, _CPS * _NP)])
    pltpu.sync_copy(acc1, s2.at[pl.ds(col0 * _NP, _CPS * _NP)])

    _init(jnp.full((16,), -jnp.inf, jnp.float32),
          jnp.full((16,), jnp.inf, jnp.float32))

    def chunk_b(k, _):
        c0 = k * _CH
        pltpu.sync_copy(srcl.at[pl.ds(c0, _CH)], esrc)
        pltpu.sync_copy(dstl.at[pl.ds(c0, _CH)], edst)

        def round1(g):
            s16 = esrc[pl.ds(g * 16, 16)]
            d16 = edst[pl.ds(g * 16, 16)]
            for c in range(_CPS):
                offd = d16 + (c * _NP)
                b = plsc.load_gather(tbl, [s16 + (c * _NP)])
                cur = plsc.load_gather(acc0, [offd])
                plsc.store_scatter(acc0, [offd], jnp.maximum(cur, b),
                                   mask=cur < b)
                curn = plsc.load_gather(acc1, [offd])
                plsc.store_scatter(acc1, [offd], jnp.minimum(curn, b),
                                   mask=curn > b)
        plsc.parallel_loop(0, _CH // 16, unroll=8)(round1)

        def verify(g):
            s16 = esrc[pl.ds(g * 16, 16)]
            d16 = edst[pl.ds(g * 16, 16)]
            pend = jnp.zeros((16,), jnp.bool_)
            for c in range(_CPS):
                offd = d16 + (c * _NP)
                b = plsc.load_gather(tbl, [s16 + (c * _NP)])
                pend = pend | (plsc.load_gather(acc0, [offd]) < b)
                pend = pend | (plsc.load_gather(acc1, [offd]) > b)
            flags[pl.ds(g * 16, 16)] = jnp.where(pend, 1, 0).astype(jnp.int32)
        plsc.parallel_loop(0, _CH // 16, unroll=8)(verify)

        def fixup(g, _):
            fv = flags[pl.ds(g * 16, 16)]

            @pl.when(jnp.max(fv) > 0)
            def _retry():
                s16 = esrc[pl.ds(g * 16, 16)]
                d16 = edst[pl.ds(g * 16, 16)]
                bs = [plsc.load_gather(tbl, [s16 + (c * _NP)])
                      for c in range(_CPS)]

                def rmw(_):
                    p2 = jnp.zeros((16,), jnp.bool_)
                    for c in range(_CPS):
                        offd = d16 + (c * _NP)
                        b = bs[c]
                        cur = plsc.load_gather(acc0, [offd])
                        plsc.store_scatter(acc0, [offd],
                                           jnp.maximum(cur, b), mask=cur < b)
                        p2 = p2 | (plsc.load_gather(acc0, [offd]) < b)
                        curn = plsc.load_gather(acc1, [offd])
                        plsc.store_scatter(acc1, [offd],
                                           jnp.minimum(curn, b), mask=curn > b)
                        p2 = p2 | (plsc.load_gather(acc1, [offd]) > b)
                    return jnp.any(p2)
                lax.while_loop(lambda p: p, rmw, jnp.bool_(True))
            return 0
        lax.fori_loop(0, _CH // 16, fixup, 0)
        return 0
    lax.fori_loop(0, _E // _CH, chunk_b, 0)
    pltpu.sync_copy(acc0, smx.at[pl.ds(col0 * _NP, _CPS * _NP)])
    pltpu.sync_copy(acc1, smn.at[pl.ds(col0 * _NP, _CPS * _NP)])

    def zcnt(j, _):
        acc0[pl.ds(j * 16, 16)] = zeros
        return 0
    lax.fori_loop(0, _NP // 16, zcnt, 0)
    ones = jnp.ones((16,), jnp.float32)
    eper = _E // _NSUB

    def chunk_c(k, _):
        c0 = wid * eper + k * _CH
        pltpu.sync_copy(dstl.at[pl.ds(c0, _CH)], edst)

        def group(g):
            plsc.addupdate_scatter(acc0, [edst[pl.ds(g * 16, 16)]], ones)
        plsc.parallel_loop(0, _CH // 16, unroll=8)(group)
        return 0
    lax.fori_loop(0, eper // _CH, chunk_c, 0)
    pltpu.sync_copy(acc0.at[pl.ds(0, _NP)], cntp.at[pl.ds(wid * _NP, _NP)])


def _post_body(ht_ref, at_ref, s1_ref, s2_ref, mx_ref, mn_ref, cntp_ref,
               pow_ref, pob_ref, lw_ref, lb_ref, out_ref):
    ht = ht_ref[...]
    a = at_ref[...]
    s1 = s1_ref[...]
    s2 = s2_ref[...]
    cnt = jnp.sum(cntp_ref[...], axis=0, keepdims=True)
    cc = jnp.maximum(cnt, 1.0)
    mean = (cnt * a + s1) / cc
    mean2 = (cnt * a * a + 2.0 * a * s1 + s2) / cc
    var = mean2 - mean * mean
    std = jnp.sqrt(jnp.maximum(var, 1e-5))
    std = jnp.where(std <= np.sqrt(1e-5), 0.0, std)
    has = cnt > 0
    mxv = jnp.where(has, a + mx_ref[...], 0.0)
    mnv = jnp.where(has, a + mn_ref[...], 0.0)
    agg = jnp.concatenate([mean, mxv, mnv, std], axis=0)
    log_deg = jnp.log(jnp.maximum(cnt, 1.0) + 1.0)
    amp = agg * (log_deg / _ADL)
    att = agg * (_ADL / log_deg)
    cat = jnp.concatenate([ht, agg, amp, att], axis=0)
    dn = (((0,), (0,)), ((), ()))
    o = (lax.dot_general(pow_ref[...], cat, dn,
                         preferred_element_type=jnp.float32) + pob_ref[...])
    o = (lax.dot_general(lw_ref[...], o, dn,
                         preferred_element_type=jnp.float32) + lb_ref[...])
    out_ref[...] = o + ht


def _post_stage(ht, at, s1, s2, mx, mn, cntp, poW, pob, lW, lb):
    fn = pl.BlockSpec((_F, _BN), lambda i: (0, i))
    return pl.pallas_call(
        _post_body,
        grid=(_NP // _BN,),
        in_specs=[fn, fn, fn, fn, fn, fn,
                  pl.BlockSpec((_NSUB, _BN), lambda i: (0, i)),
                  pl.BlockSpec((13 * _F, _F), lambda i: (0, 0)),
                  pl.BlockSpec((_F, 1), lambda i: (0, 0)),
                  pl.BlockSpec((_F, _F), lambda i: (0, 0)),
                  pl.BlockSpec((_F, 1), lambda i: (0, 0))],
        out_specs=fn,
        out_shape=jax.ShapeDtypeStruct((_F, _NP), jnp.float32),
    )(ht, at, s1, s2, mx, mn, cntp, poW, pob[:, None], lW, lb[:, None])


def kernel(x, edge_index, pre_W, pre_b, post_W, post_b, lin_W, lin_b):
    src, dst = edge_index[0], edge_index[1]
    ht = jnp.pad(x.T, ((0, 0), (0, _NP - _N)))
    for i in range(_LAYERS):
        at, bt = _pre_stage(ht, pre_W[i], pre_b[i])
        s1, s2, smx, smn, cntp = _sc_edge(bt.reshape(_F * _NP), src, dst)
        ht = _post_stage(ht, at,
                         s1.reshape(_F, _NP), s2.reshape(_F, _NP),
                         smx.reshape(_F, _NP), smn.reshape(_F, _NP),
                         cntp.reshape(_NSUB, _NP),
                         post_W[i], post_b[i], lin_W[i], lin_b[i])
    return ht[:, :_N].T


# speedup vs baseline: 3.1041x; 1.2384x over previous
import functools

import jax
import jax.numpy as jnp
import numpy as np
from jax import lax
from jax.experimental import pallas as pl
from jax.experimental.pallas import tpu as pltpu, tpu_sc as plsc

_N = 10000
_NP = 10240
_E = 320000
_F = 128
_LAYERS = 3
_ADL = 0.0

_BN = 1024
_CH = 2000
_CPS = 4
_NSUB = 32

_sc_mesh = plsc.VectorSubcoreMesh(core_axis_name="c", subcore_axis_name="s")


def _pre_body(ht_ref, wd_ref, ws_ref, pb_ref, at_ref, bt_ref):
    ht = ht_ref[...]
    dn = (((0,), (0,)), ((), ()))
    at_ref[...] = (lax.dot_general(wd_ref[...], ht, dn,
                                   preferred_element_type=jnp.float32)
                   + pb_ref[...])
    bt_ref[...] = lax.dot_general(ws_ref[...], ht, dn,
                                  preferred_element_type=jnp.float32)


def _pre_stage(ht, pW, pb):
    wd, ws = pW[:_F], pW[_F:]
    fn = pl.BlockSpec((_F, _BN), lambda i: (0, i))
    ff = pl.BlockSpec((_F, _F), lambda i: (0, 0))
    return pl.pallas_call(
        _pre_body,
        grid=(_NP // _BN,),
        in_specs=[fn, ff, ff, pl.BlockSpec((_F, 1), lambda i: (0, 0))],
        out_specs=[fn, fn],
        out_shape=[jax.ShapeDtypeStruct((_F, _NP), jnp.float32),
                   jax.ShapeDtypeStruct((_F, _NP), jnp.float32)],
    )(ht, wd, ws, pb[:, None])


@functools.partial(
    pl.kernel, mesh=_sc_mesh,
    compiler_params=pltpu.CompilerParams(needs_layout_passes=False),
    out_type=[jax.ShapeDtypeStruct((_F * _NP,), jnp.float32),
              jax.ShapeDtypeStruct((_F * _NP,), jnp.float32),
              jax.ShapeDtypeStruct((_F * _NP,), jnp.float32),
              jax.ShapeDtypeStruct((_F * _NP,), jnp.float32),
              jax.ShapeDtypeStruct((_NSUB * _NP,), jnp.float32)],
    scratch_types=[pltpu.VMEM((_CPS * _NP,), jnp.float32),
                   pltpu.VMEM((_CPS * _NP,), jnp.float32),
                   pltpu.VMEM((_CPS * _NP,), jnp.float32),
                   pltpu.VMEM((_CH,), jnp.int32),
                   pltpu.VMEM((_CH,), jnp.int32),
                   pltpu.VMEM((_CH,), jnp.int32)])
def _sc_edge(bt, srcl, dstl, s1, s2, smx, smn, cntp,
             tbl, acc0, acc1, esrc, edst, flags):
    cid = lax.axis_index("c")
    sid = lax.axis_index("s")
    wid = sid * 2 + cid
    col0 = wid * _CPS

    pltpu.sync_copy(bt.at[pl.ds(col0 * _NP, _CPS * _NP)], tbl)

    zeros = jnp.zeros((16,), jnp.float32)

    def _init(val0, val1):
        def body(j, _):
            acc0[pl.ds(j * 16, 16)] = val0
            acc1[pl.ds(j * 16, 16)] = val1
            return 0
        lax.fori_loop(0, (_CPS * _NP) // 16, body, 0)

    _init(zeros, zeros)

    def chunk_a(k, _):
        c0 = k * _CH
        pltpu.sync_copy(srcl.at[pl.ds(c0, _CH)], esrc)
        pltpu.sync_copy(dstl.at[pl.ds(c0, _CH)], edst)

        def group(g):
            s16 = esrc[pl.ds(g * 16, 16)]
            d16 = edst[pl.ds(g * 16, 16)]
            for c in range(_CPS):
                b = plsc.load_gather(tbl, [s16 + (c * _NP)])
                offd = d16 + (c * _NP)
                plsc.addupdate_scatter(acc0, [offd], b)
                plsc.addupdate_scatter(acc1, [offd], b * b)
        plsc.parallel_loop(0, _CH // 16, unroll=8)(group)
        return 0
    lax.fori_loop(0, _E // _CH, chunk_a, 0)
    pltpu.sync_copy(acc0, s1.at[pl.ds(col0 * _NP, _CPS * _NP)])
    pltpu.sync_copy(acc1, s2.at[pl.ds(col0 * _NP, _CPS * _NP)])

    _init(jnp.full((16,), -jnp.inf, jnp.float32),
          jnp.full((16,), jnp.inf, jnp.float32))

    def chunk_b(k, _):
        c0 = k * _CH
        pltpu.sync_copy(srcl.at[pl.ds(c0, _CH)], esrc)
        pltpu.sync_copy(dstl.at[pl.ds(c0, _CH)], edst)

        def round1(g):
            s16 = esrc[pl.ds(g * 16, 16)]
            d16 = edst[pl.ds(g * 16, 16)]
            for c in range(_CPS):
                offd = d16 + (c * _NP)
                b = plsc.load_gather(tbl, [s16 + (c * _NP)])
                cur = plsc.load_gather(acc0, [offd])
                plsc.store_scatter(acc0, [offd], jnp.maximum(cur, b),
                                   mask=cur < b)
                curn = plsc.load_gather(acc1, [offd])
                plsc.store_scatter(acc1, [offd], jnp.minimum(curn, b),
                                   mask=curn > b)
        plsc.parallel_loop(0, _CH // 16, unroll=8)(round1)

        def verify(g):
            s16 = esrc[pl.ds(g * 16, 16)]
            d16 = edst[pl.ds(g * 16, 16)]
            pend = jnp.zeros((16,), jnp.bool_)
            for c in range(_CPS):
                offd = d16 + (c * _NP)
                b = plsc.load_gather(tbl, [s16 + (c * _NP)])
                pend = pend | (plsc.load_gather(acc0, [offd]) < b)
                pend = pend | (plsc.load_gather(acc1, [offd]) > b)
            flags[pl.ds(g * 16, 16)] = jnp.where(pend, 1, 0).astype(jnp.int32)
        plsc.parallel_loop(0, _CH // 16, unroll=8)(verify)

        def fixup_group(g):
            fv = flags[pl.ds(g * 16, 16)]

            @pl.when(jnp.max(fv) > 0)
            def _retry():
                s16 = esrc[pl.ds(g * 16, 16)]
                d16 = edst[pl.ds(g * 16, 16)]
                bs = [plsc.load_gather(tbl, [s16 + (c * _NP)])
                      for c in range(_CPS)]

                def rmw(_):
                    p2 = jnp.zeros((16,), jnp.bool_)
                    for c in range(_CPS):
                        offd = d16 + (c * _NP)
                        b = bs[c]
                        cur = plsc.load_gather(acc0, [offd])
                        plsc.store_scatter(acc0, [offd],
                                           jnp.maximum(cur, b), mask=cur < b)
                        p2 = p2 | (plsc.load_gather(acc0, [offd]) < b)
                        curn = plsc.load_gather(acc1, [offd])
                        plsc.store_scatter(acc1, [offd],
                                           jnp.minimum(curn, b), mask=curn > b)
                        p2 = p2 | (plsc.load_gather(acc1, [offd]) > b)
                    return jnp.any(p2)
                lax.while_loop(lambda p: p, rmw, jnp.bool_(True))

        def fixup5(q, _):
            g0 = q * 5
            fv = flags[pl.ds(g0 * 16, 16)]
            for j in range(1, 5):
                fv = jnp.maximum(fv, flags[pl.ds((g0 + j) * 16, 16)])

            @pl.when(jnp.max(fv) > 0)
            def _drill():
                for j in range(5):
                    fixup_group(g0 + j)
            return 0
        lax.fori_loop(0, _CH // 80, fixup5, 0)
        return 0
    lax.fori_loop(0, _E // _CH, chunk_b, 0)
    pltpu.sync_copy(acc0, smx.at[pl.ds(col0 * _NP, _CPS * _NP)])
    pltpu.sync_copy(acc1, smn.at[pl.ds(col0 * _NP, _CPS * _NP)])

    def zcnt(j, _):
        acc0[pl.ds(j * 16, 16)] = zeros
        return 0
    lax.fori_loop(0, _NP // 16, zcnt, 0)
    ones = jnp.ones((16,), jnp.float32)
    eper = _E // _NSUB

    def chunk_c(k, _):
        c0 = wid * eper + k * _CH
        pltpu.sync_copy(dstl.at[pl.ds(c0, _CH)], edst)

        def group(g):
            plsc.addupdate_scatter(acc0, [edst[pl.ds(g * 16, 16)]], ones)
        plsc.parallel_loop(0, _CH // 16, unroll=8)(group)
        return 0
    lax.fori_loop(0, eper // _CH, chunk_c, 0)
    pltpu.sync_copy(acc0.at[pl.ds(0, _NP)], cntp.at[pl.ds(wid * _NP, _NP)])


def _post_body(ht_ref, at_ref, s1_ref, s2_ref, mx_ref, mn_ref, cntp_ref,
               pow_ref, pob_ref, lw_ref, lb_ref, out_ref):
    ht = ht_ref[...]
    a = at_ref[...]
    s1 = s1_ref[...]
    s2 = s2_ref[...]
    cnt = jnp.sum(cntp_ref[...], axis=0, keepdims=True)
    cc = jnp.maximum(cnt, 1.0)
    mean = (cnt * a + s1) / cc
    mean2 = (cnt * a * a + 2.0 * a * s1 + s2) / cc
    var = mean2 - mean * mean
    std = jnp.sqrt(jnp.maximum(var, 1e-5))
    std = jnp.where(std <= np.sqrt(1e-5), 0.0, std)
    has = cnt > 0
    mxv = jnp.where(has, a + mx_ref[...], 0.0)
    mnv = jnp.where(has, a + mn_ref[...], 0.0)
    agg = jnp.concatenate([mean, mxv, mnv, std], axis=0)
    log_deg = jnp.log(jnp.maximum(cnt, 1.0) + 1.0)
    amp = agg * (log_deg / _ADL)
    att = agg * (_ADL / log_deg)
    cat = jnp.concatenate([ht, agg, amp, att], axis=0)
    dn = (((0,), (0,)), ((), ()))
    o = (lax.dot_general(pow_ref[...], cat, dn,
                         preferred_element_type=jnp.float32) + pob_ref[...])
    o = (lax.dot_general(lw_ref[...], o, dn,
                         preferred_element_type=jnp.float32) + lb_ref[...])
    out_ref[...] = o + ht


def _post_stage(ht, at, s1, s2, mx, mn, cntp, poW, pob, lW, lb):
    fn = pl.BlockSpec((_F, _BN), lambda i: (0, i))
    return pl.pallas_call(
        _post_body,
        grid=(_NP // _BN,),
        in_specs=[fn, fn, fn, fn, fn, fn,
                  pl.BlockSpec((_NSUB, _BN), lambda i: (0, i)),
                  pl.BlockSpec((13 * _F, _F), lambda i: (0, 0)),
                  pl.BlockSpec((_F, 1), lambda i: (0, 0)),
                  pl.BlockSpec((_F, _F), lambda i: (0, 0)),
                  pl.BlockSpec((_F, 1), lambda i: (0, 0))],
        out_specs=fn,
        out_shape=jax.ShapeDtypeStruct((_F, _NP), jnp.float32),
    )(ht, at, s1, s2, mx, mn, cntp, poW, pob[:, None], lW, lb[:, None])


def kernel(x, edge_index, pre_W, pre_b, post_W, post_b, lin_W, lin_b):
    src, dst = edge_index[0], edge_index[1]
    ht = jnp.pad(x.T, ((0, 0), (0, _NP - _N)))
    for i in range(_LAYERS):
        at, bt = _pre_stage(ht, pre_W[i], pre_b[i])
        s1, s2, smx, smn, cntp = _sc_edge(bt.reshape(_F * _NP), src, dst)
        ht = _post_stage(ht, at,
                         s1.reshape(_F, _NP), s2.reshape(_F, _NP),
                         smx.reshape(_F, _NP), smn.reshape(_F, _NP),
                         cntp.reshape(_NSUB, _NP),
                         post_W[i], post_b[i], lin_W[i], lin_b[i])
    return ht[:, :_N].T
